# baseline (speedup 1.0000x reference)
import jax
import jax.numpy as jnp
from jax import lax
from jax.experimental import pallas as pl
from jax.experimental.pallas import tpu as pltpu
from jax.experimental.pallas import tpu_sc as plsc

N_USER = 25000
N_ITEM = 25000
N_NODES = N_USER + N_ITEM
EMB = 64
HALF = EMB // 2
N_EDGES = 800000
N_TILES = 16
EDGES_PER_TILE = N_EDGES // N_TILES
BLK_EDGES = 2560
FULL_BLOCKS = EDGES_PER_TILE // BLK_EDGES
REM_EDGES = EDGES_PER_TILE - FULL_BLOCKS * BLK_EDGES
EOP = 256
OPS_PER_BLK = BLK_EDGES // EOP
REM_OPS = REM_EDGES // EOP
REM_TAIL = REM_EDGES - REM_OPS * EOP
NBUF = 5
PASSES = 2
QCOL = HALF // PASSES
N_PAD = 50176
ROWS_PER_TILE = N_PAD // N_TILES
ROW_BLK = 112
ROW_ITERS = ROWS_PER_TILE // ROW_BLK
USER_TAIL = N_USER % ROW_BLK
ITEM_TAIL = N_NODES % ROW_BLK


def _sc_body(user_e, item_e, eidx, aval, out_user, out_item, out_l, out_ego,
             src_t, acc, cbi, cbv, gb0, gb1, gb2, gb3, gb4,
             gs0, gs1, gs2, gs3, gs4, ss0, ss1, ss2, ss3, ss4):
    gbs = (gb0, gb1, gb2, gb3, gb4)
    gss = (gs0, gs1, gs2, gs3, gs4)
    sss = (ss0, ss1, ss2, ss3, ss4)
    c = lax.axis_index("c")
    t = lax.axis_index("s")
    cf = c * HALF

    zv = jnp.zeros((16,), jnp.float32)

    for p in range(PASSES):
        if p > 0:
            plsc.subcore_barrier()
        pcol = p * QCOL
        qc0 = cf + pcol

        @plsc.parallel_loop(0, ROW_BLK)
        def zrow(r):
            gb0[r, pl.ds(0, 16)] = zv

        def zcopy(i, _):
            rbase = t * ROWS_PER_TILE + i * ROW_BLK
            e0 = rbase
            e1 = rbase + ROW_BLK
            pltpu.sync_copy(gb0.at[pl.ds(0, ROW_BLK)],
                            acc.at[pl.ds(rbase, ROW_BLK)])
            my_blk = lax.rem(i, 2) == c

            @pl.when(e1 <= N_USER)
            def _():
                pltpu.sync_copy(
                    user_e.at[pl.ds(rbase, ROW_BLK), pl.ds(qc0, QCOL)],
                    src_t.at[pl.ds(rbase, ROW_BLK)])
                if p == 0:
                    @pl.when(my_blk)
                    def _():
                        pltpu.sync_copy(user_e.at[pl.ds(rbase, ROW_BLK)],
                                        out_ego.at[pl.ds(rbase, ROW_BLK)])

            @pl.when(jnp.logical_and(e0 < N_USER, e1 > N_USER))
            def _():
                pltpu.sync_copy(
                    user_e.at[pl.ds(rbase, USER_TAIL), pl.ds(qc0, QCOL)],
                    src_t.at[pl.ds(rbase, USER_TAIL)])
                pltpu.sync_copy(
                    item_e.at[pl.ds(0, ROW_BLK - USER_TAIL),
                              pl.ds(qc0, QCOL)],
                    src_t.at[pl.ds(rbase + USER_TAIL, ROW_BLK - USER_TAIL)])
                if p == 0:
                    @pl.when(my_blk)
                    def _():
                        pltpu.sync_copy(user_e.at[pl.ds(rbase, USER_TAIL)],
                                        out_ego.at[pl.ds(rbase, USER_TAIL)])
                        pltpu.sync_copy(
                            item_e.at[pl.ds(0, ROW_BLK - USER_TAIL)],
                            out_ego.at[pl.ds(rbase + USER_TAIL,
                                             ROW_BLK - USER_TAIL)])

            @pl.when(jnp.logical_and(e0 >= N_USER, e1 <= N_NODES))
            def _():
                pltpu.sync_copy(
                    item_e.at[pl.ds(rbase - N_USER, ROW_BLK),
                              pl.ds(qc0, QCOL)],
                    src_t.at[pl.ds(rbase, ROW_BLK)])
                if p == 0:
                    @pl.when(my_blk)
                    def _():
                        pltpu.sync_copy(
                            item_e.at[pl.ds(rbase - N_USER, ROW_BLK)],
                            out_ego.at[pl.ds(rbase, ROW_BLK)])

            @pl.when(jnp.logical_and(e0 < N_NODES, e1 > N_NODES))
            def _():
                pltpu.sync_copy(
                    item_e.at[pl.ds(rbase - N_USER, ITEM_TAIL),
                              pl.ds(qc0, QCOL)],
                    src_t.at[pl.ds(rbase, ITEM_TAIL)])
                if p == 0:
                    @pl.when(my_blk)
                    def _():
                        pltpu.sync_copy(
                            item_e.at[pl.ds(rbase - N_USER, ITEM_TAIL)],
                            out_ego.at[pl.ds(rbase, ITEM_TAIL)])

            return 0

        lax.fori_loop(0, ROW_ITERS, zcopy, 0)
        plsc.subcore_barrier()

        def run_ops(n_ops, eop):
            def quad(q, _):
                eq = q * (NBUF * eop)
                gd = [pltpu.async_copy(
                    src_t.at[cbi.at[1, pl.ds(eq + i * eop, eop)]],
                    gbs[i].at[pl.ds(0, eop)],
                    gss[i]) for i in range(NBUF)]
                sd = []
                for i in range(NBUF):
                    gd[i].wait()
                    gbuf = gbs[i]
                    e0 = eq + i * eop

                    @plsc.parallel_loop(0, eop // 16)
                    def edge_group(g, gbuf=gbuf, e0=e0):
                        vg = cbv[pl.ds(e0 + g * 16, 16)]
                        for e in range(16):
                            v = vg[e]
                            r = g * 16 + e
                            gbuf[r, pl.ds(0, 16)] = gbuf[r, pl.ds(0, 16)] * v

                    sd.append(pltpu.async_copy(
                        gbuf.at[pl.ds(0, eop)],
                        acc.at[cbi.at[0, pl.ds(e0, eop)]],
                        sss[i], add=True))
                for d in sd:
                    d.wait()
                return 0

            lax.fori_loop(0, n_ops // NBUF, quad, 0)

        def outer(o, _):
            ebase = t * EDGES_PER_TILE + o * BLK_EDGES
            pltpu.sync_copy(eidx.at[:, pl.ds(ebase, BLK_EDGES)], cbi)
            pltpu.sync_copy(aval.at[pl.ds(ebase, BLK_EDGES)], cbv)
            run_ops(OPS_PER_BLK, EOP)
            return 0

        lax.fori_loop(0, FULL_BLOCKS, outer, 0)

        rbase_e = t * EDGES_PER_TILE + FULL_BLOCKS * BLK_EDGES
        pltpu.sync_copy(eidx.at[0, pl.ds(rbase_e, REM_EDGES)],
                        cbi.at[0, pl.ds(0, REM_EDGES)])
        pltpu.sync_copy(eidx.at[1, pl.ds(rbase_e, REM_EDGES)],
                        cbi.at[1, pl.ds(0, REM_EDGES)])
        pltpu.sync_copy(aval.at[pl.ds(rbase_e, REM_EDGES)],
                        cbv.at[pl.ds(0, REM_EDGES)])
        run_ops(REM_OPS, EOP)
        te = REM_OPS * EOP
        pltpu.async_copy(src_t.at[cbi.at[1, pl.ds(te, REM_TAIL)]],
                         gb0.at[pl.ds(0, REM_TAIL)], gs0).wait()

        @plsc.parallel_loop(0, REM_TAIL // 16)
        def tail_group(g):
            vg = cbv[pl.ds(te + g * 16, 16)]
            for e in range(16):
                v = vg[e]
                r = g * 16 + e
                gb0[r, pl.ds(0, 16)] = gb0[r, pl.ds(0, 16)] * v

        pltpu.async_copy(gb0.at[pl.ds(0, REM_TAIL)],
                         acc.at[cbi.at[0, pl.ds(te, REM_TAIL)]],
                         ss0, add=True).wait()
        plsc.subcore_barrier()

        qc = cf + pcol

        def out_iter(i, _):
            rbase = t * ROWS_PER_TILE + i * ROW_BLK
            e0 = rbase
            e1 = rbase + ROW_BLK
            tb = gb0.at[pl.ds(0, ROW_BLK)]
            eb = gb1.at[pl.ds(0, ROW_BLK)]
            pltpu.sync_copy(acc.at[pl.ds(rbase, ROW_BLK)], tb)
            pltpu.sync_copy(src_t.at[pl.ds(rbase, ROW_BLK)], eb)

            @plsc.parallel_loop(0, ROW_BLK)
            def crow(r):
                sl = pl.ds(0, 16)
                gb1[r, sl] = (gb1[r, sl] + 3.0 * gb0[r, sl]) * 0.25

            @pl.when(e1 <= N_NODES)
            def _():
                pltpu.sync_copy(tb, out_l.at[pl.ds(rbase, ROW_BLK),
                                             pl.ds(qc, QCOL)])

            @pl.when(jnp.logical_and(e1 > N_NODES, e0 < N_NODES))
            def _():
                pltpu.sync_copy(gb0.at[pl.ds(0, ITEM_TAIL)],
                                out_l.at[pl.ds(rbase, ITEM_TAIL),
                                         pl.ds(qc, QCOL)])

            @pl.when(e1 <= N_USER)
            def _():
                pltpu.sync_copy(gb1.at[pl.ds(0, ROW_BLK)],
                                out_user.at[pl.ds(rbase, ROW_BLK),
                                            pl.ds(qc, QCOL)])

            @pl.when(jnp.logical_and(e0 < N_USER, e1 > N_USER))
            def _():
                pltpu.sync_copy(gb1.at[pl.ds(0, USER_TAIL)],
                                out_user.at[pl.ds(rbase, USER_TAIL),
                                            pl.ds(qc, QCOL)])
                pltpu.sync_copy(gb1.at[pl.ds(USER_TAIL, ROW_BLK - USER_TAIL)],
                                out_item.at[pl.ds(0, ROW_BLK - USER_TAIL),
                                            pl.ds(qc, QCOL)])

            @pl.when(jnp.logical_and(e0 >= N_USER, e1 <= N_NODES))
            def _():
                pltpu.sync_copy(gb1.at[pl.ds(0, ROW_BLK)],
                                out_item.at[pl.ds(rbase - N_USER, ROW_BLK),
                                            pl.ds(qc, QCOL)])

            @pl.when(jnp.logical_and(e0 < N_NODES, e1 > N_NODES))
            def _():
                pltpu.sync_copy(gb1.at[pl.ds(0, ITEM_TAIL)],
                                out_item.at[pl.ds(rbase - N_USER, ITEM_TAIL),
                                            pl.ds(qc, QCOL)])

            return 0

        lax.fori_loop(0, ROW_ITERS, out_iter, 0)


@jax.jit
def _propagate(user_e, item_e, eidx, aval):
    mesh = plsc.VectorSubcoreMesh(core_axis_name="c", subcore_axis_name="s")
    fn = pl.kernel(
        _sc_body,
        out_type=(
            jax.ShapeDtypeStruct((N_USER, EMB), jnp.float32),
            jax.ShapeDtypeStruct((N_ITEM, EMB), jnp.float32),
            jax.ShapeDtypeStruct((N_NODES, EMB), jnp.float32),
            jax.ShapeDtypeStruct((N_NODES, EMB), jnp.float32),
        ),
        mesh=mesh,
        compiler_params=pltpu.CompilerParams(use_tc_tiling_on_sc=False,
                                             needs_layout_passes=False),
        scratch_types=[
            pltpu.VMEM_SHARED((N_PAD, QCOL), jnp.float32),
            pltpu.VMEM_SHARED((N_PAD, QCOL), jnp.float32),
            pltpu.VMEM((2, BLK_EDGES), jnp.int32),
            pltpu.VMEM((BLK_EDGES,), jnp.float32),
            pltpu.VMEM((EOP, QCOL), jnp.float32),
            pltpu.VMEM((EOP, QCOL), jnp.float32),
            pltpu.VMEM((EOP, QCOL), jnp.float32),
            pltpu.VMEM((EOP, QCOL), jnp.float32),
            pltpu.VMEM((EOP, QCOL), jnp.float32),
            pltpu.SemaphoreType.DMA,
            pltpu.SemaphoreType.DMA,
            pltpu.SemaphoreType.DMA,
            pltpu.SemaphoreType.DMA,
            pltpu.SemaphoreType.DMA,
            pltpu.SemaphoreType.DMA,
            pltpu.SemaphoreType.DMA,
            pltpu.SemaphoreType.DMA,
            pltpu.SemaphoreType.DMA,
            pltpu.SemaphoreType.DMA,
        ],
    )
    return fn(user_e, item_e, eidx, aval)


def kernel(user_emb, item_emb, edge_index, adj_val):
    user_all, item_all, layer, ego = _propagate(
        user_emb.astype(jnp.float32), item_emb.astype(jnp.float32),
        edge_index.astype(jnp.int32), adj_val.astype(jnp.float32))
    return (user_all, item_all, ego, layer, layer, layer)
